# baseline (speedup 1.0000x reference)
import functools

import jax
import jax.numpy as jnp
from jax.experimental import pallas as pl
from jax.experimental.pallas import tpu as pltpu

_MARGIN = 0.2


_TOL = 1e18


def _bisect(keys, k, lo, hi, axis, n_iter=None):
    def step(carry):
        lo, hi, c_lo = carry
        mid = lo + ((hi - lo + 1) >> 1)
        cnt = jnp.sum((keys >= mid).astype(jnp.int32), axis=axis, keepdims=True)
        ge = cnt >= k
        return (jnp.where(ge, mid, lo), jnp.where(ge, hi, mid - 1),
                jnp.where(ge, cnt, c_lo))

    c0 = jnp.full(lo.shape, keys.shape[axis], jnp.int32)
    if n_iter is not None:
        lo, hi, _ = jax.lax.fori_loop(
            0, n_iter, lambda _, c: step(c), (lo, hi, c0))
    else:
        def cond(c):
            lo, hi, c_lo = c
            width = (hi - lo).astype(jnp.float32)
            extra = (c_lo - k).astype(jnp.float32)
            return jnp.any(width * extra > _TOL)

        lo, hi, _ = jax.lax.while_loop(cond, step, (lo, hi, c0))
    return lo


def _topk_sum(keys, k, lo, hi, axis):
    t_bits = _bisect(keys, k, lo, hi, axis, n_iter=13)
    t = jax.lax.bitcast_convert_type(t_bits, jnp.float32)
    gt = keys > t_bits
    vals = jax.lax.bitcast_convert_type(keys, jnp.float32)
    s = jnp.sum(jnp.where(gt, vals, 0.0), axis=axis, keepdims=True)
    c = jnp.sum(gt.astype(jnp.float32), axis=axis, keepdims=True)
    return s + (k.astype(jnp.float32) - c) * t


def _row_body(neg_ref, x_ref, o_ref, keys_ref, *, block: int):
    i = pl.program_id(0)
    R = block
    x = x_ref[...]
    xd = x_ref[:, pl.ds(i * R, R)]
    rr = jax.lax.broadcasted_iota(jnp.int32, (R, R), 0)
    cc = jax.lax.broadcasted_iota(jnp.int32, (R, R), 1)
    deq = rr == cc
    d = jnp.sum(jnp.where(deq, xd, 0.0), axis=1, keepdims=True)
    cost = jnp.maximum(x + (_MARGIN - d), 0.0)
    keys_ref[...] = jax.lax.bitcast_convert_type(cost, jnp.int32)
    dblk = keys_ref[:, pl.ds(i * R, R)]
    keys_ref[:, pl.ds(i * R, R)] = jnp.where(deq, 0, dblk)
    keys = keys_ref[...]
    k = neg_ref[0]

    m = keys
    w = m.shape[1]
    while w > 128:
        w //= 2
        m = jnp.maximum(m[:, :w], m[:, w:])
    rowmax = jnp.max(m, axis=1, keepdims=True)
    zero = jnp.zeros((R, 1), jnp.int32)
    tau = _bisect(m, k, zero, rowmax, axis=1, n_iter=31)
    o_ref[...] = _topk_sum(keys, k, tau, rowmax, axis=1)


def _col_body(neg_ref, x_ref, o_ref, keys_ref, *, block: int):
    j = pl.program_id(0)
    C = block
    x = x_ref[...]
    xd = x_ref[pl.ds(j * C, C), :]
    rr = jax.lax.broadcasted_iota(jnp.int32, (C, C), 0)
    cc = jax.lax.broadcasted_iota(jnp.int32, (C, C), 1)
    deq = rr == cc
    d = jnp.sum(jnp.where(deq, xd, 0.0), axis=0, keepdims=True)
    cost = jnp.maximum(x + (_MARGIN - d), 0.0)
    keys_ref[...] = jax.lax.bitcast_convert_type(cost, jnp.int32)
    dblk = keys_ref[pl.ds(j * C, C), :]
    keys_ref[pl.ds(j * C, C), :] = jnp.where(deq, 0, dblk)
    keys = keys_ref[...]
    k = neg_ref[0]

    m = keys
    w = m.shape[0]
    while w > 128:
        w //= 2
        m = jnp.maximum(m[:w, :], m[w:, :])
    colmax = jnp.max(m, axis=0, keepdims=True)
    zero = jnp.zeros((1, C), jnp.int32)
    tau = _bisect(m, k, zero, colmax, axis=0, n_iter=31)
    res = _topk_sum(keys, k, tau, colmax, axis=0)
    o_ref[...] = jnp.broadcast_to(res, o_ref.shape)


def _run(scores, neg, *, block: int = 256, interpret: bool = False):
    n = scores.shape[0]
    neg_arr = jnp.asarray(neg, jnp.int32).reshape(1)
    grid = (n // block,)

    row_out = pl.pallas_call(
        functools.partial(_row_body, block=block),
        grid=grid,
        in_specs=[
            pl.BlockSpec(memory_space=pltpu.SMEM),
            pl.BlockSpec((block, n), lambda i: (i, 0)),
        ],
        out_specs=pl.BlockSpec((block, 1), lambda i: (i, 0)),
        out_shape=jax.ShapeDtypeStruct((n, 1), jnp.float32),
        scratch_shapes=[pltpu.VMEM((block, n), jnp.int32)],
        interpret=interpret,
    )(neg_arr, scores)

    col_out = pl.pallas_call(
        functools.partial(_col_body, block=block),
        grid=grid,
        in_specs=[
            pl.BlockSpec(memory_space=pltpu.SMEM),
            pl.BlockSpec((n, block), lambda j: (0, j)),
        ],
        out_specs=pl.BlockSpec((8, block), lambda j: (0, j)),
        out_shape=jax.ShapeDtypeStruct((8, n), jnp.float32),
        scratch_shapes=[pltpu.VMEM((n, block), jnp.int32)],
        interpret=interpret,
    )(neg_arr, scores)

    return (row_out[:, 0] + col_out[0, :]) / neg


def kernel(scores, neg):
    return _run(scores, neg)
